# Initial kernel scaffold; baseline (speedup 1.0000x reference)
import jax
import jax.numpy as jnp
from jax.experimental import pallas as pl


def kernel(X, emb_table):
    raise NotImplementedError("write your pallas kernel here")


# speedup vs baseline: 1.5123x; 1.5123x over previous
import functools

import jax
import jax.numpy as jnp
from jax import lax
from jax.experimental import pallas as pl
from jax.experimental.pallas import tpu as pltpu
from jax.experimental.pallas import tpu_sc as plsc

VOCAB = 1000000
DIM = 32
PAD = 0

NC = 2
NS = 16
L = 16
NW = NC * NS

B = 4096 * 200
BPW = B // NW
K = BPW // 128
CH = 1280
NSUB = CH // 128
NB = BPW // CH


def _body(table_hbm, idx_hbm, out_hbm, idx_v, rows_v, sem):
    wid = lax.axis_index("s") * NC + lax.axis_index("c")
    base = wid * BPW

    pltpu.sync_copy(idx_hbm.at[wid], idx_v)

    iota = lax.iota(jnp.int32, L)
    zrow = jnp.zeros((L,), jnp.float32)

    def chunk(g, carry):
        copies = []
        for j in range(NSUB):
            copies.append(
                pltpu.async_copy(
                    table_hbm.at[idx_v.at[g * NSUB + j]],
                    rows_v.at[pl.ds(j * 128, 128)],
                    sem,
                )
            )
        for c in copies:
            c.wait()

        def scan128(i8, carry2):
            for k in range(8):
                iv = idx_v[g * NSUB + i8, pl.ds(k * 16, 16)]
                m = iv == PAD

                @pl.when(jnp.min(iv) == PAD)
                def _():
                    row = (i8 * 128 + k * 16) + iota
                    for col in range(DIM):
                        plsc.store_scatter(
                            rows_v,
                            [row, jnp.full((L,), col, jnp.int32)],
                            zrow,
                            mask=m,
                        )

            return carry2

        lax.fori_loop(0, NSUB, scan128, 0)

        pltpu.sync_copy(rows_v, out_hbm.at[pl.ds(base + g * CH, CH)])
        return carry

    lax.fori_loop(0, NB, chunk, 0)


@functools.partial(jax.jit, static_argnames=())
def _embed(table, idx3):
    f = pl.kernel(
        _body,
        out_type=jax.ShapeDtypeStruct((B, DIM), jnp.float32),
        mesh=plsc.VectorSubcoreMesh(
            core_axis_name="c", subcore_axis_name="s",
            num_cores=NC, num_subcores=NS,
        ),
        scratch_types=[
            pltpu.VMEM((K, 128), jnp.int32),
            pltpu.VMEM((CH, DIM), jnp.float32),
            pltpu.SemaphoreType.DMA,
        ],
        compiler_params=pltpu.CompilerParams(
            use_tc_tiling_on_sc=False, needs_layout_passes=False,
        ),
    )
    return f(table, idx3)


def kernel(X, emb_table):
    idx3 = X.reshape(NW, K, 128).astype(jnp.int32)
    out = _embed(emb_table, idx3)
    return out.reshape(X.shape[0], X.shape[1], DIM)


# speedup vs baseline: 1.5805x; 1.0451x over previous
import functools

import jax
import jax.numpy as jnp
from jax import lax
from jax.experimental import pallas as pl
from jax.experimental.pallas import tpu as pltpu
from jax.experimental.pallas import tpu_sc as plsc

VOCAB = 1000000
DIM = 32
PAD = 0

NC = 2
NS = 16
L = 16
NW = NC * NS

B = 4096 * 200
BPW = B // NW
K = BPW // 128
CH = 1280
NSUB = CH // 128
NB = BPW // CH


def _body(table_hbm, idx_hbm, out_hbm, idx_v, rows2, gsem, osem):
    wid = lax.axis_index("s") * NC + lax.axis_index("c")
    base = wid * BPW

    pltpu.sync_copy(idx_hbm.at[wid], idx_v)

    iota = lax.iota(jnp.int32, L)
    zrow = jnp.zeros((L,), jnp.float32)

    def fire_gathers(g, b):
        for j in range(NSUB):
            pltpu.async_copy(
                table_hbm.at[idx_v.at[g * NSUB + j]],
                rows2.at[b].at[pl.ds(j * 128, 128)],
                gsem.at[b],
            )

    def drain_gathers(g, b):
        pltpu.make_async_copy(
            out_hbm.at[pl.ds(base + g * CH, CH)],
            rows2.at[b],
            gsem.at[b],
        ).wait()

    def fire_out(g, b):
        pltpu.async_copy(
            rows2.at[b], out_hbm.at[pl.ds(base + g * CH, CH)], osem.at[b]
        )

    def wait_out(b):
        pltpu.make_async_copy(
            rows2.at[b], out_hbm.at[pl.ds(base, CH)], osem.at[b]
        ).wait()

    def pad_scan(g, b):
        rows_v = rows2.at[b]

        def scan128(i8, carry2):
            vs = [idx_v[g * NSUB + i8, pl.ds(k * 16, 16)] for k in range(8)]
            mn = vs[0]
            for k in range(1, 8):
                mn = jnp.minimum(mn, vs[k])

            @pl.when(jnp.min(mn) == PAD)
            def _():
                for k in range(8):
                    m = vs[k] == PAD

                    @pl.when(jnp.min(vs[k]) == PAD)
                    def _():
                        row = (i8 * 128 + k * 16) + iota
                        for col in range(DIM):
                            plsc.store_scatter(
                                rows_v,
                                [row, jnp.full((L,), col, jnp.int32)],
                                zrow,
                                mask=m,
                            )

            return carry2

        lax.fori_loop(0, NSUB, scan128, 0)

    fire_gathers(0, 0)

    def block(t, carry):
        g0 = 2 * t
        for b in range(2):
            g = g0 + b
            nb = 1 - b

            @pl.when(g + 1 < NB)
            def _():
                @pl.when(g > 0)
                def _():
                    wait_out(nb)

                fire_gathers(g + 1, nb)

            drain_gathers(g, b)
            pad_scan(g, b)
            fire_out(g, b)
        return carry

    lax.fori_loop(0, NB // 2, block, 0)
    wait_out(0)
    wait_out(1)


@functools.partial(jax.jit, static_argnames=())
def _embed(table, idx3):
    f = pl.kernel(
        _body,
        out_type=jax.ShapeDtypeStruct((B, DIM), jnp.float32),
        mesh=plsc.VectorSubcoreMesh(
            core_axis_name="c", subcore_axis_name="s",
            num_cores=NC, num_subcores=NS,
        ),
        scratch_types=[
            pltpu.VMEM((K, 128), jnp.int32),
            pltpu.VMEM((2, CH, DIM), jnp.float32),
            pltpu.SemaphoreType.DMA((2,)),
            pltpu.SemaphoreType.DMA((2,)),
        ],
        compiler_params=pltpu.CompilerParams(
            use_tc_tiling_on_sc=False, needs_layout_passes=False,
        ),
    )
    return f(table, idx3)


def kernel(X, emb_table):
    idx3 = X.reshape(NW, K, 128).astype(jnp.int32)
    out = _embed(emb_table, idx3)
    return out.reshape(X.shape[0], X.shape[1], DIM)
